# speedup vs baseline: 2.5144x; 2.5144x over previous
import functools

import jax
import jax.numpy as jnp
from jax import lax
from jax.experimental import pallas as pl
from jax.experimental.pallas import tpu as pltpu
from jax.experimental.pallas import tpu_sc as plsc

D = 128
N = 4096 * 50
NW = 32
PER_W = N // NW
CHUNK = 128
G = PER_W // CHUNK
LANES = 16
KSL = D // LANES


def _sc_body(preds_hbm, idx_hbm, table_hbm, out_hbm,
             idx_v, preds_v, rows_v, acc_v, sem_p, sem_g):
    c = lax.axis_index("c")
    s = lax.axis_index("s")
    wid = s * 2 + c
    pltpu.sync_copy(idx_hbm.at[wid], idx_v)
    base = wid * PER_W

    def chunk_body(g, accs):
        cb = pl.multiple_of(base + g * CHUNK, CHUNK)
        cp = pltpu.async_copy(preds_hbm.at[pl.ds(cb, CHUNK)], preds_v, sem_p)
        cg = pltpu.async_copy(table_hbm.at[idx_v.at[g]], rows_v, sem_g)
        cp.wait()
        cg.wait()

        def row_body(r, a):
            new = []
            for k in range(KSL):
                p = preds_v[r, pl.ds(k * LANES, LANES)]
                e = rows_v[r, pl.ds(k * LANES, LANES)]
                d = p - e
                new.append(a[k] + d * d)
            return tuple(new)

        return lax.fori_loop(0, CHUNK, row_body, accs)

    accs = tuple(jnp.zeros((LANES,), jnp.float32) for _ in range(KSL))
    accs = lax.fori_loop(0, G, chunk_body, accs)
    total = accs[0]
    for k in range(1, KSL):
        total = total + accs[k]
    acc_v[...] = total
    pltpu.sync_copy(acc_v, out_hbm.at[wid])


@jax.jit
def _sc_partials(preds2, idx, table):
    mesh = plsc.VectorSubcoreMesh(core_axis_name="c", subcore_axis_name="s")
    f = functools.partial(
        pl.kernel,
        mesh=mesh,
        out_type=jax.ShapeDtypeStruct((NW, LANES), jnp.float32),
        scratch_types=[
            pltpu.VMEM((G, CHUNK), jnp.int32),
            pltpu.VMEM((CHUNK, D), jnp.float32),
            pltpu.VMEM((CHUNK, D), jnp.float32),
            pltpu.VMEM((LANES,), jnp.float32),
            pltpu.SemaphoreType.DMA,
            pltpu.SemaphoreType.DMA,
        ],
    )(_sc_body)
    return f(preds2, idx, table)


def kernel(preds, target, table):
    idx = target.astype(jnp.int32).reshape(NW, G, CHUNK)
    preds2 = preds.reshape(N, D)
    partials = _sc_partials(preds2, idx, table)
    return jnp.sum(partials) / jnp.float32(N * D)


# speedup vs baseline: 3.0885x; 1.2283x over previous
import functools

import jax
import jax.numpy as jnp
from jax import lax
from jax.experimental import pallas as pl
from jax.experimental.pallas import tpu as pltpu
from jax.experimental.pallas import tpu_sc as plsc

D = 128
N = 4096 * 50
NW = 32
PER_W = N // NW
CHUNK = 128
G = PER_W // CHUNK
LANES = 16
KSL = D // LANES


UNROLL = 4


def _sc_body(preds_hbm, idx_hbm, table_hbm, out_hbm,
             idx_v, preds_v0, rows_v0, preds_v1, rows_v1, acc_v,
             sem_p0, sem_g0, sem_p1, sem_g1):
    c = lax.axis_index("c")
    s = lax.axis_index("s")
    wid = s * 2 + c
    pltpu.sync_copy(idx_hbm.at[wid], idx_v)
    base = wid * PER_W

    bufs = ((preds_v0, rows_v0, sem_p0, sem_g0),
            (preds_v1, rows_v1, sem_p1, sem_g1))

    def start(g, buf):
        preds_v, rows_v, sem_p, sem_g = buf
        cb = pl.multiple_of(base + g * CHUNK, CHUNK)
        pltpu.async_copy(preds_hbm.at[pl.ds(cb, CHUNK)], preds_v, sem_p)
        pltpu.async_copy(table_hbm.at[idx_v.at[g]], rows_v, sem_g)

    def wait(buf):
        preds_v, rows_v, sem_p, sem_g = buf
        pltpu.make_async_copy(preds_hbm.at[pl.ds(0, CHUNK)], preds_v, sem_p).wait()
        pltpu.make_async_copy(preds_hbm.at[pl.ds(0, CHUNK)], rows_v, sem_g).wait()

    def compute(buf, accs):
        preds_v, rows_v, _, _ = buf

        def row_body(r, a):
            for u in range(UNROLL):
                ru = r * UNROLL + u
                new = []
                for k in range(KSL):
                    p = preds_v[ru, pl.ds(k * LANES, LANES)]
                    e = rows_v[ru, pl.ds(k * LANES, LANES)]
                    d = p - e
                    new.append(a[k] + d * d)
                a = tuple(new)
            return a

        return lax.fori_loop(0, CHUNK // UNROLL, row_body, accs)

    start(0, bufs[0])
    start(1, bufs[1])
    H = G // 2

    def pair_body(h, accs):
        wait(bufs[0])
        accs = compute(bufs[0], accs)

        @pl.when(h + 1 < H)
        def _():
            start(2 * h + 2, bufs[0])

        wait(bufs[1])
        accs = compute(bufs[1], accs)

        @pl.when(h + 1 < H)
        def _():
            start(2 * h + 3, bufs[1])

        return accs

    accs = tuple(jnp.zeros((LANES,), jnp.float32) for _ in range(KSL))
    accs = lax.fori_loop(0, H, pair_body, accs)
    total = accs[0]
    for k in range(1, KSL):
        total = total + accs[k]
    acc_v[...] = total
    pltpu.sync_copy(acc_v, out_hbm.at[wid])


@jax.jit
def _sc_partials(preds2, idx, table):
    mesh = plsc.VectorSubcoreMesh(core_axis_name="c", subcore_axis_name="s")
    f = functools.partial(
        pl.kernel,
        mesh=mesh,
        out_type=jax.ShapeDtypeStruct((NW, LANES), jnp.float32),
        scratch_types=[
            pltpu.VMEM((G, CHUNK), jnp.int32),
            pltpu.VMEM((CHUNK, D), jnp.float32),
            pltpu.VMEM((CHUNK, D), jnp.float32),
            pltpu.VMEM((CHUNK, D), jnp.float32),
            pltpu.VMEM((CHUNK, D), jnp.float32),
            pltpu.VMEM((LANES,), jnp.float32),
            pltpu.SemaphoreType.DMA,
            pltpu.SemaphoreType.DMA,
            pltpu.SemaphoreType.DMA,
            pltpu.SemaphoreType.DMA,
        ],
    )(_sc_body)
    return f(preds2, idx, table)


def kernel(preds, target, table):
    idx = target.astype(jnp.int32).reshape(NW, G, CHUNK)
    preds2 = preds.reshape(N, D)
    partials = _sc_partials(preds2, idx, table)
    return jnp.sum(partials) / jnp.float32(N * D)


# speedup vs baseline: 4.9839x; 1.6137x over previous
import functools

import jax
import jax.numpy as jnp
from jax import lax
from jax.experimental import pallas as pl
from jax.experimental.pallas import tpu as pltpu
from jax.experimental.pallas import tpu_sc as plsc

B = 4096
S = 50
D = 128
NW = 32
EPW = B // NW
CE = 2
G = EPW // CE
H = G // 2
LANES = 16
KSL = D // LANES
RUNROLL = 2


def _sc_body(preds_hbm, idx_hbm, table_hbm, out_hbm,
             idx_all, preds_v0, rows_v0, preds_v1, rows_v1, acc_v,
             sem_p0, sem_g0, sem_p1, sem_g1):
    c = lax.axis_index("c")
    s = lax.axis_index("s")
    wid = s * 2 + c
    base = wid * EPW
    pltpu.sync_copy(idx_hbm.at[pl.ds(base, EPW)], idx_all)

    bufs = ((preds_v0, rows_v0, sem_p0, sem_g0),
            (preds_v1, rows_v1, sem_p1, sem_g1))

    def start(g, buf):
        preds_v, rows_v, sem_p, sem_g = buf
        eb = pl.multiple_of(base + g * CE, CE)
        pltpu.async_copy(preds_hbm.at[pl.ds(eb, CE)], preds_v, sem_p)
        for j in range(CE):
            e = g * CE + j
            pltpu.async_copy(table_hbm.at[idx_all.at[e]], rows_v.at[j], sem_g)

    def wait_data(buf):
        preds_v, rows_v, sem_p, sem_g = buf
        pltpu.make_async_copy(preds_hbm.at[pl.ds(0, CE)], preds_v, sem_p).wait()
        pltpu.make_async_copy(preds_hbm.at[pl.ds(0, CE)], rows_v, sem_g).wait()

    def compute(buf, accs):
        preds_v, rows_v, _, _ = buf

        def elem(j, accs):
            def row_body(r, a):
                for u in range(RUNROLL):
                    ru = r * RUNROLL + u
                    new = []
                    for k in range(KSL):
                        p = preds_v[j, ru, pl.ds(k * LANES, LANES)]
                        e = rows_v[j, ru, pl.ds(k * LANES, LANES)]
                        d = p - e
                        new.append(a[k] + d * d)
                    a = tuple(new)
                return a

            return lax.fori_loop(0, S // RUNROLL, row_body, accs)

        for j in range(CE):
            accs = elem(j, accs)
        return accs

    start(0, bufs[0])
    start(1, bufs[1])

    def chunk_pair(h, accs):
        wait_data(bufs[0])
        accs = compute(bufs[0], accs)

        @pl.when(h + 1 < H)
        def _():
            start(2 * h + 2, bufs[0])

        wait_data(bufs[1])
        accs = compute(bufs[1], accs)

        @pl.when(h + 1 < H)
        def _():
            start(2 * h + 3, bufs[1])

        return accs

    accs = tuple(jnp.zeros((LANES,), jnp.float32) for _ in range(KSL))
    accs = lax.fori_loop(0, H, chunk_pair, accs)
    total = accs[0]
    for k in range(1, KSL):
        total = total + accs[k]
    acc_v[...] = total
    pltpu.sync_copy(acc_v, out_hbm.at[wid])


@jax.jit
def _sc_partials(preds, target, table):
    mesh = plsc.VectorSubcoreMesh(core_axis_name="c", subcore_axis_name="s")
    f = functools.partial(
        pl.kernel,
        mesh=mesh,
        out_type=jax.ShapeDtypeStruct((NW, LANES), jnp.float32),
        scratch_types=[
            pltpu.VMEM((EPW, S), jnp.int32),
            pltpu.VMEM((CE, S, D), jnp.float32),
            pltpu.VMEM((CE, S, D), jnp.float32),
            pltpu.VMEM((CE, S, D), jnp.float32),
            pltpu.VMEM((CE, S, D), jnp.float32),
            pltpu.VMEM((LANES,), jnp.float32),
            pltpu.SemaphoreType.DMA,
            pltpu.SemaphoreType.DMA,
            pltpu.SemaphoreType.DMA,
            pltpu.SemaphoreType.DMA,
        ],
    )(_sc_body)
    return f(preds, target, table)


def kernel(preds, target, table):
    partials = _sc_partials(preds, target, table)
    return jnp.sum(partials) / jnp.float32(B * S * D)


# speedup vs baseline: 4.9982x; 1.0029x over previous
import functools

import jax
import jax.numpy as jnp
from jax import lax
from jax.experimental import pallas as pl
from jax.experimental.pallas import tpu as pltpu
from jax.experimental.pallas import tpu_sc as plsc

B = 4096
S = 50
D = 128
NW = 32
EPW = B // NW
CE = 2
G = EPW // CE
H = G // 2
LANES = 16
KSL = D // LANES
RUNROLL = 2


def _sc_body(preds_hbm, idx_hbm, table_hbm, out_hbm,
             idx_all, preds_v0, rows_v0, preds_v1, rows_v1, acc_v,
             sem_p0, sem_g0, sem_p1, sem_g1):
    c = lax.axis_index("c")
    s = lax.axis_index("s")
    wid = s * 2 + c
    base = wid * EPW
    pltpu.sync_copy(idx_hbm.at[pl.ds(base, EPW)], idx_all)

    bufs = ((preds_v0, rows_v0, sem_p0, sem_g0),
            (preds_v1, rows_v1, sem_p1, sem_g1))

    def start(g, buf):
        preds_v, rows_v, sem_p, sem_g = buf
        eb = pl.multiple_of(base + g * CE, CE)
        pltpu.async_copy(preds_hbm.at[pl.ds(eb, CE)], preds_v, sem_p)
        for j in range(CE):
            e = g * CE + j
            pltpu.async_copy(table_hbm.at[idx_all.at[e]], rows_v.at[j], sem_g)

    def wait_data(buf):
        preds_v, rows_v, sem_p, sem_g = buf
        pltpu.make_async_copy(preds_hbm.at[pl.ds(0, CE)], preds_v, sem_p).wait()
        pltpu.make_async_copy(preds_hbm.at[pl.ds(0, CE)], rows_v, sem_g).wait()

    def compute(buf, accs):
        preds_v, rows_v, _, _ = buf

        def elem(j, accs):
            def row_body(r, a):
                for u in range(RUNROLL):
                    ru = r * RUNROLL + u
                    new = []
                    for k in range(KSL):
                        p = preds_v[j, ru, pl.ds(k * LANES, LANES)]
                        e = rows_v[j, ru, pl.ds(k * LANES, LANES)]
                        d = p - e
                        new.append(a[k] + d * d)
                    a = tuple(new)
                return a

            return lax.fori_loop(0, S // RUNROLL, row_body, accs)

        for j in range(CE):
            accs = elem(j, accs)
        return accs

    start(0, bufs[0])
    start(1, bufs[1])

    def chunk_pair(h, accs):
        wait_data(bufs[0])
        accs = compute(bufs[0], accs)

        @pl.when(h + 1 < H)
        def _():
            start(2 * h + 2, bufs[0])

        wait_data(bufs[1])
        accs = compute(bufs[1], accs)

        @pl.when(h + 1 < H)
        def _():
            start(2 * h + 3, bufs[1])

        return accs

    accs = tuple(jnp.zeros((LANES,), jnp.float32) for _ in range(KSL))
    accs = lax.fori_loop(0, H, chunk_pair, accs)
    total = accs[0]
    for k in range(1, KSL):
        total = total + accs[k]
    acc_v[...] = total
    pltpu.sync_copy(acc_v, out_hbm.at[wid])


@jax.jit
def _sc_partials(preds, target, table):
    mesh = plsc.VectorSubcoreMesh(core_axis_name="c", subcore_axis_name="s")
    f = functools.partial(
        pl.kernel,
        mesh=mesh,
        compiler_params=pltpu.CompilerParams(use_tc_tiling_on_sc=True),
        out_type=jax.ShapeDtypeStruct((NW, LANES), jnp.float32),
        scratch_types=[
            pltpu.VMEM((EPW, S), jnp.int32),
            pltpu.VMEM((CE, S, D), jnp.float32),
            pltpu.VMEM((CE, S, D), jnp.float32),
            pltpu.VMEM((CE, S, D), jnp.float32),
            pltpu.VMEM((CE, S, D), jnp.float32),
            pltpu.VMEM((LANES,), jnp.float32),
            pltpu.SemaphoreType.DMA,
            pltpu.SemaphoreType.DMA,
            pltpu.SemaphoreType.DMA,
            pltpu.SemaphoreType.DMA,
        ],
    )(_sc_body)
    return f(preds, target, table)


def kernel(preds, target, table):
    partials = _sc_partials(preds, target, table)
    return jnp.sum(partials) / jnp.float32(B * S * D)


# speedup vs baseline: 8.2169x; 1.6440x over previous
import functools

import jax
import jax.numpy as jnp
from jax import lax
from jax.experimental import pallas as pl
from jax.experimental.pallas import tpu as pltpu
from jax.experimental.pallas import tpu_sc as plsc

B = 4096
S = 50
D = 128
NW = 32
BW = B // NW
LANES = 16
KSL = D // LANES
RUNROLL = 2


def _sc_body(preds_hbm, idx_hbm, table_hbm, out_hbm,
             idx_all, preds_v0, rows_v0, preds_v1, rows_v1, acc_v,
             sem_p0, sem_g0, sem_p1, sem_g1):
    c = lax.axis_index("c")
    s = lax.axis_index("s")
    wid = s * 2 + c
    bb = pl.multiple_of(wid * BW, BW)
    pltpu.sync_copy(idx_hbm.at[:, pl.ds(bb, BW)], idx_all)

    bufs = ((preds_v0, rows_v0, sem_p0, sem_g0),
            (preds_v1, rows_v1, sem_p1, sem_g1))

    def start(g, buf):
        preds_v, rows_v, sem_p, sem_g = buf
        pltpu.async_copy(preds_hbm.at[g, pl.ds(bb, BW)], preds_v, sem_p)
        pltpu.async_copy(table_hbm.at[idx_all.at[g]], rows_v, sem_g)

    def wait_data(buf):
        preds_v, rows_v, sem_p, sem_g = buf
        pltpu.make_async_copy(preds_hbm.at[0, pl.ds(0, BW)], preds_v, sem_p).wait()
        pltpu.make_async_copy(preds_hbm.at[0, pl.ds(0, BW)], rows_v, sem_g).wait()

    def compute(buf, accs):
        preds_v, rows_v, _, _ = buf

        def row_body(r, a):
            for u in range(RUNROLL):
                ru = r * RUNROLL + u
                new = []
                for k in range(KSL):
                    p = preds_v[ru, pl.ds(k * LANES, LANES)]
                    e = rows_v[ru, pl.ds(k * LANES, LANES)]
                    d = p - e
                    new.append(a[k] + d * d)
                a = tuple(new)
            return a

        return lax.fori_loop(0, BW // RUNROLL, row_body, accs)

    start(0, bufs[0])
    start(1, bufs[1])
    H = S // 2

    def chunk_pair(h, accs):
        wait_data(bufs[0])
        accs = compute(bufs[0], accs)

        @pl.when(h + 1 < H)
        def _():
            start(2 * h + 2, bufs[0])

        wait_data(bufs[1])
        accs = compute(bufs[1], accs)

        @pl.when(h + 1 < H)
        def _():
            start(2 * h + 3, bufs[1])

        return accs

    accs = tuple(jnp.zeros((LANES,), jnp.float32) for _ in range(KSL))
    accs = lax.fori_loop(0, H, chunk_pair, accs)
    total = accs[0]
    for k in range(1, KSL):
        total = total + accs[k]
    acc_v[...] = total
    pltpu.sync_copy(acc_v, out_hbm.at[wid])


@jax.jit
def _sc_partials(preds_t, target_t, table):
    mesh = plsc.VectorSubcoreMesh(core_axis_name="c", subcore_axis_name="s")
    f = functools.partial(
        pl.kernel,
        mesh=mesh,
        out_type=jax.ShapeDtypeStruct((NW, LANES), jnp.float32),
        scratch_types=[
            pltpu.VMEM((S, BW), jnp.int32),
            pltpu.VMEM((BW, D), jnp.float32),
            pltpu.VMEM((BW, D), jnp.float32),
            pltpu.VMEM((BW, D), jnp.float32),
            pltpu.VMEM((BW, D), jnp.float32),
            pltpu.VMEM((LANES,), jnp.float32),
            pltpu.SemaphoreType.DMA,
            pltpu.SemaphoreType.DMA,
            pltpu.SemaphoreType.DMA,
            pltpu.SemaphoreType.DMA,
        ],
    )(_sc_body)
    return f(preds_t, target_t, table)


def kernel(preds, target, table):
    preds_t = jnp.transpose(preds, (1, 0, 2))
    target_t = jnp.transpose(target, (1, 0))
    partials = _sc_partials(preds_t, target_t, table)
    return jnp.sum(partials) / jnp.float32(B * S * D)


# speedup vs baseline: 9.5913x; 1.1673x over previous
import functools

import jax
import jax.numpy as jnp
from jax import lax
from jax.experimental import pallas as pl
from jax.experimental.pallas import tpu as pltpu
from jax.experimental.pallas import tpu_sc as plsc

B = 4096
S = 50
D = 128
NW = 32
BW = B // NW
CR = 64
NCH = S * BW // CR
NBUF = 4
LANES = 16
KSL = D // LANES
RUNROLL = 4


def _sc_body(preds_hbm, idx_hbm, table_hbm, out_hbm,
             idx_all, preds_v0, rows_v0, preds_v1, rows_v1,
             preds_v2, rows_v2, preds_v3, rows_v3, acc_v,
             sem_p0, sem_g0, sem_p1, sem_g1,
             sem_p2, sem_g2, sem_p3, sem_g3):
    c = lax.axis_index("c")
    s = lax.axis_index("s")
    wid = s * 2 + c
    bb = pl.multiple_of(wid * BW, BW)
    pltpu.sync_copy(idx_hbm.at[:, pl.ds(bb, BW)], idx_all)

    bufs = ((preds_v0, rows_v0, sem_p0, sem_g0),
            (preds_v1, rows_v1, sem_p1, sem_g1),
            (preds_v2, rows_v2, sem_p2, sem_g2),
            (preds_v3, rows_v3, sem_p3, sem_g3))

    def start(g, buf):
        preds_v, rows_v, sem_p, sem_g = buf
        sq = g // 2
        off = pl.multiple_of(bb + (g % 2) * CR, CR)
        pltpu.async_copy(preds_hbm.at[sq, pl.ds(off, CR)], preds_v, sem_p)
        pltpu.async_copy(
            table_hbm.at[idx_all.at[sq, pl.ds((g % 2) * CR, CR)]], rows_v, sem_g)

    def wait_data(buf):
        preds_v, rows_v, sem_p, sem_g = buf
        pltpu.make_async_copy(preds_hbm.at[0, pl.ds(0, CR)], preds_v, sem_p).wait()
        pltpu.make_async_copy(preds_hbm.at[0, pl.ds(0, CR)], rows_v, sem_g).wait()

    def compute(buf, accs):
        preds_v, rows_v, _, _ = buf

        def row_body(r, a):
            for u in range(RUNROLL):
                ru = r * RUNROLL + u
                new = []
                for k in range(KSL):
                    p = preds_v[ru, pl.ds(k * LANES, LANES)]
                    e = rows_v[ru, pl.ds(k * LANES, LANES)]
                    d = p - e
                    new.append(a[k] + d * d)
                a = tuple(new)
            return a

        return lax.fori_loop(0, CR // RUNROLL, row_body, accs)

    for b in range(NBUF):
        start(b, bufs[b])
    H = NCH // NBUF

    def ring_body(h, accs):
        for b in range(NBUF):
            g = h * NBUF + b
            wait_data(bufs[b])
            accs = compute(bufs[b], accs)

            @pl.when(g + NBUF < NCH)
            def _():
                start(g + NBUF, bufs[b])

        return accs

    accs = tuple(jnp.zeros((LANES,), jnp.float32) for _ in range(KSL))
    accs = lax.fori_loop(0, H, ring_body, accs)
    total = accs[0]
    for k in range(1, KSL):
        total = total + accs[k]
    acc_v[...] = total
    pltpu.sync_copy(acc_v, out_hbm.at[wid])


@jax.jit
def _sc_partials(preds_t, target_t, table):
    mesh = plsc.VectorSubcoreMesh(core_axis_name="c", subcore_axis_name="s")
    f = functools.partial(
        pl.kernel,
        mesh=mesh,
        out_type=jax.ShapeDtypeStruct((NW, LANES), jnp.float32),
        scratch_types=(
            [pltpu.VMEM((S, BW), jnp.int32)]
            + [pltpu.VMEM((CR, D), jnp.float32) for _ in range(2 * NBUF)]
            + [pltpu.VMEM((LANES,), jnp.float32)]
            + [pltpu.SemaphoreType.DMA for _ in range(2 * NBUF)]
        ),
    )(_sc_body)
    return f(preds_t, target_t, table)


def kernel(preds, target, table):
    preds_t = jnp.transpose(preds, (1, 0, 2))
    target_t = jnp.transpose(target, (1, 0))
    partials = _sc_partials(preds_t, target_t, table)
    return jnp.sum(partials) / jnp.float32(B * S * D)
